# Initial kernel scaffold; baseline (speedup 1.0000x reference)
import jax
import jax.numpy as jnp
from jax.experimental import pallas as pl


def kernel(x, y_idx, s, b):
    raise NotImplementedError("write your pallas kernel here")


# speedup vs baseline: 5.4923x; 5.4923x over previous
import functools

import jax
import jax.numpy as jnp
from jax import lax
from jax.experimental import pallas as pl
from jax.experimental.pallas import tpu as pltpu

E = 8
D = 1024
N = 32768
BLK = 2048


def _affine_body(y_ref, s_ref, b_ref, x_ref, out_ref, ld_ref):
    y = y_ref[0, 0, :]
    classes = lax.broadcasted_iota(jnp.int32, (1, E), 1)
    onehot = (y[:, None] == classes).astype(jnp.float32)
    s_full = s_ref[...]
    es = jnp.exp(s_full)
    es_tok = jnp.dot(onehot, es, preferred_element_type=jnp.float32)
    b_tok = jnp.dot(onehot, b_ref[...], preferred_element_type=jnp.float32)
    out_ref[...] = x_ref[...] * es_tok + b_tok
    row_sums = jnp.sum(s_full, axis=1)
    ld_ref[0, 0, :] = jnp.sum(onehot * row_sums[None, :], axis=1)


@jax.jit
def kernel(x, y_idx, s, b):
    nblk = N // BLK
    y3 = y_idx.astype(jnp.int32).reshape(nblk, 1, BLK)
    out, ld3 = pl.pallas_call(
        _affine_body,
        grid=(nblk,),
        in_specs=[
            pl.BlockSpec((1, 1, BLK), lambda i: (i, 0, 0)),
            pl.BlockSpec((E, D), lambda i: (0, 0)),
            pl.BlockSpec((E, D), lambda i: (0, 0)),
            pl.BlockSpec((BLK, D), lambda i: (i, 0)),
        ],
        out_specs=[
            pl.BlockSpec((BLK, D), lambda i: (i, 0)),
            pl.BlockSpec((1, 1, BLK), lambda i: (i, 0, 0)),
        ],
        out_shape=[
            jax.ShapeDtypeStruct((N, D), jnp.float32),
            jax.ShapeDtypeStruct((nblk, 1, BLK), jnp.float32),
        ],
        compiler_params=pltpu.CompilerParams(
            dimension_semantics=("arbitrary",),
        ),
    )(y3, s, b, x)
    return out, ld3.reshape(N)
